# Initial kernel scaffold; baseline (speedup 1.0000x reference)
import jax
import jax.numpy as jnp
from jax.experimental import pallas as pl


def kernel(user_x, edge_index, W_l0, b_l0, W_r0, W_l1, b_l1, W_r1, gamma0, beta0, gamma1, beta1, ded_table, bil_W, bil_b):
    raise NotImplementedError("write your pallas kernel here")


# speedup vs baseline: 4.5238x; 4.5238x over previous
import functools

import jax
import jax.numpy as jnp
from jax import lax
from jax.experimental import pallas as pl
from jax.experimental.pallas import tpu as pltpu
from jax.experimental.pallas import tpu_sc as plsc

N_NODES = 10000
D_IN = 128
D_H = 64
N_EDGES = 320000

NC = 2
NS = 16
NW = NC * NS
ROW = 128
G = 4
ROWS_PER_TILE = 80
E_PAD = NW * ROWS_PER_TILE * ROW
N_PAD = 10240
N_PER_TILE = N_PAD // NS
DEG_W = 16

@functools.lru_cache(maxsize=None)
def _make_edge_pass(with_deg: bool):
  out_type = [jax.ShapeDtypeStruct((NC, N_PAD, D_H), jnp.float32)]
  scratch = [
      pltpu.VMEM_SHARED((N_PAD, D_H), jnp.float32),
      pltpu.VMEM((G, ROW), jnp.int32),
      pltpu.VMEM((G, ROW), jnp.int32),
      pltpu.VMEM((ROW, D_H), jnp.float32),
      pltpu.VMEM((ROW, D_H), jnp.float32),
  ]
  if with_deg:
    out_type.append(jax.ShapeDtypeStruct((NC, N_PAD, DEG_W), jnp.float32))
    scratch.append(pltpu.VMEM_SHARED((N_PAD, DEG_W), jnp.float32))
    scratch.append(pltpu.VMEM((ROW, DEG_W), jnp.float32))
    scratch.append(pltpu.VMEM((ROW, DEG_W), jnp.float32))

  def body(p_hbm, src_hbm, dst_hbm, acc_out, *rest):
    if with_deg:
      (deg_out, acc_sh, src_v, dst_v, rows_v, zbuf_v,
       deg_sh, ones_v, zdeg_v) = rest
    else:
      acc_sh, src_v, dst_v, rows_v, zbuf_v = rest
    c = lax.axis_index("c")
    s = lax.axis_index("s")

    @pl.loop(0, ROW)
    def _(r):
      for j in range(D_H // 16):
        zbuf_v[r, pl.ds(j * 16, 16)] = jnp.zeros((16,), jnp.float32)
      if with_deg:
        ones_v[r, pl.ds(0, 16)] = jnp.ones((16,), jnp.float32)
        zdeg_v[r, pl.ds(0, 16)] = jnp.zeros((16,), jnp.float32)

    for k in range(N_PER_TILE // ROW):
      r0 = s * N_PER_TILE + k * ROW
      pltpu.sync_copy(zbuf_v, acc_sh.at[pl.ds(r0, ROW)])
      if with_deg:
        pltpu.sync_copy(zdeg_v, deg_sh.at[pl.ds(r0, ROW)])
    plsc.subcore_barrier()

    wid = s * NC + c
    row0 = wid * ROWS_PER_TILE

    @pl.loop(0, ROWS_PER_TILE // G)
    def _(g):
      r = row0 + g * G
      pltpu.sync_copy(src_hbm.at[pl.ds(r, G)], src_v)
      pltpu.sync_copy(dst_hbm.at[pl.ds(r, G)], dst_v)
      for j in range(G):
        pltpu.sync_copy(p_hbm.at[src_v.at[j]], rows_v)
        pltpu.sync_copy(rows_v, acc_sh.at[dst_v.at[j]], add=True)
        if with_deg:
          pltpu.sync_copy(ones_v, deg_sh.at[dst_v.at[j]], add=True)
    plsc.subcore_barrier()

    nr = s * N_PER_TILE
    pltpu.sync_copy(acc_sh.at[pl.ds(nr, N_PER_TILE)],
                    acc_out.at[c, pl.ds(nr, N_PER_TILE)])
    if with_deg:
      pltpu.sync_copy(deg_sh.at[pl.ds(nr, N_PER_TILE)],
                      deg_out.at[c, pl.ds(nr, N_PER_TILE)])

  mesh = plsc.VectorSubcoreMesh(core_axis_name="c", subcore_axis_name="s",
                                num_cores=NC, num_subcores=NS)
  return pl.kernel(
      body, out_type=out_type, mesh=mesh, scratch_types=scratch,
      compiler_params=pltpu.CompilerParams(use_tc_tiling_on_sc=False))


def _edge_pass_deg(p, src_p, dst_p):
  return _make_edge_pass(True)(p, src_p, dst_p)


def _edge_pass(p, src_p, dst_p):
  return _make_edge_pass(False)(p, src_p, dst_p)


def _dot(a, b):
  return lax.dot_general(a, b, (((1,), (0,)), ((), ())),
                         precision=lax.Precision.HIGHEST,
                         preferred_element_type=jnp.float32)


def _proj2_body(x_ref, wl_ref, wr_ref, p_ref, r_ref):
  x = x_ref[...]
  p_ref[...] = _dot(x, wl_ref[...])
  r_ref[...] = _dot(x, wr_ref[...])


def _proj2(x, wl, wr):
  n = x.shape[0]
  return pl.pallas_call(
      _proj2_body,
      out_shape=(jax.ShapeDtypeStruct((n, D_H), jnp.float32),
                 jax.ShapeDtypeStruct((n, D_H), jnp.float32)),
  )(x, wl, wr)


def _bn_relu(h, gamma, beta):
  mu = jnp.mean(h, axis=0, keepdims=True)
  var = jnp.mean(jnp.square(h - mu), axis=0, keepdims=True)
  hn = (h - mu) * lax.rsqrt(var + 1e-5) * gamma + beta
  return jnp.maximum(hn, 0.0)


def _bn_stage_body(acc_ref, deg_ref, r_ref, b_ref, g_ref, be_ref, h_ref):
  agg = acc_ref[0, :N_NODES, :] + acc_ref[1, :N_NODES, :]
  deg = deg_ref[0, :N_NODES, 0:1] + deg_ref[1, :N_NODES, 0:1]
  h = agg / jnp.maximum(deg, 1.0) + b_ref[...] + r_ref[...]
  h_ref[...] = _bn_relu(h, g_ref[...], be_ref[...])


def _bn_stage(acc, deg, r, b, gamma, beta):
  return pl.pallas_call(
      _bn_stage_body,
      out_shape=jax.ShapeDtypeStruct((N_NODES, D_H), jnp.float32),
  )(acc, deg, r, b, gamma, beta)


def _score_body(h_ref, ded_ref, bw_ref, bb_ref, out_ref):
  hw = _dot(h_ref[...], bw_ref[...])
  scores = lax.dot_general(hw, ded_ref[...], (((1,), (1,)), ((), ())),
                           precision=lax.Precision.HIGHEST,
                           preferred_element_type=jnp.float32)
  out_ref[...] = jax.nn.sigmoid(scores + bb_ref[0, 0])


def _score(h2, ded_table, bil_W, bil_b):
  return pl.pallas_call(
      _score_body,
      out_shape=jax.ShapeDtypeStruct((N_NODES, N_DED_OUT), jnp.float32),
  )(h2, ded_table, bil_W, bil_b)


N_DED_OUT = 8


def kernel(user_x, edge_index, W_l0, b_l0, W_r0, W_l1, b_l1, W_r1,
           gamma0, beta0, gamma1, beta1, ded_table, bil_W, bil_b):
  src = edge_index[0]
  dst = edge_index[1]
  npad = E_PAD - N_EDGES
  src_p = jnp.concatenate([src, jnp.zeros((npad,), jnp.int32)])
  dst_p = jnp.concatenate([dst, jnp.full((npad,), N_NODES, jnp.int32)])
  src_p = src_p.reshape(E_PAD // ROW, ROW)
  dst_p = dst_p.reshape(E_PAD // ROW, ROW)
  b_l0_ = b_l0.reshape(1, D_H)
  b_l1_ = b_l1.reshape(1, D_H)
  gamma0_ = gamma0.reshape(1, D_H)
  beta0_ = beta0.reshape(1, D_H)
  gamma1_ = gamma1.reshape(1, D_H)
  beta1_ = beta1.reshape(1, D_H)
  bil_b_ = bil_b.reshape(1, 1)

  p0, r0 = _proj2(user_x, W_l0, W_r0)
  acc0, deg = _edge_pass_deg(p0, src_p, dst_p)
  h1 = _bn_stage(acc0, deg, r0, b_l0_, gamma0_, beta0_)
  p1, r1 = _proj2(h1, W_l1, W_r1)
  (acc1,) = _edge_pass(p1, src_p, dst_p)
  h2 = _bn_stage(acc1, deg, r1, b_l1_, gamma1_, beta1_)
  return _score(h2, ded_table, bil_W, bil_b_)


# speedup vs baseline: 5.0045x; 1.1063x over previous
import functools

import jax
import jax.numpy as jnp
from jax import lax
from jax.experimental import pallas as pl
from jax.experimental.pallas import tpu as pltpu
from jax.experimental.pallas import tpu_sc as plsc

N_NODES = 10000
D_IN = 128
D_H = 64
N_EDGES = 320000
N_DED_OUT = 8

NC = 2
NS = 16
NW = NC * NS
ROW = 128
K = 2
ROWS_PER_TILE = 80
GROUPS = ROWS_PER_TILE // K
PAIRS = GROUPS // 2
E_PAD = NW * ROWS_PER_TILE * ROW
N_PAD = 10240
N_PER_TILE = N_PAD // NS
D_AUG = D_H + 16


@functools.lru_cache(maxsize=None)
def _make_edge_pass(width: int):
  out_type = jax.ShapeDtypeStruct((NC, N_PAD, width), jnp.float32)
  scratch = [
      pltpu.VMEM_SHARED((N_PAD, width), jnp.float32),
      pltpu.VMEM((ROWS_PER_TILE, 2, ROW), jnp.int32),
      pltpu.VMEM((K * ROW, width), jnp.float32),
      pltpu.VMEM((K * ROW, width), jnp.float32),
      pltpu.VMEM((ROW, width), jnp.float32),
      pltpu.SemaphoreType.DMA,
      pltpu.SemaphoreType.DMA,
      pltpu.SemaphoreType.DMA,
      pltpu.SemaphoreType.DMA,
  ]

  def body(p_hbm, idx_hbm, acc_out, acc_sh, idx_v, rows_a, rows_b, zbuf_v,
           gsem_a, gsem_b, ssem_a, ssem_b):
    c = lax.axis_index("c")
    s = lax.axis_index("s")

    @pl.loop(0, ROW)
    def _(r):
      for j in range(width // 16):
        zbuf_v[r, pl.ds(j * 16, 16)] = jnp.zeros((16,), jnp.float32)

    for k in range(N_PER_TILE // ROW):
      pltpu.sync_copy(zbuf_v, acc_sh.at[pl.ds(s * N_PER_TILE + k * ROW, ROW)])

    wid = s * NC + c
    pltpu.sync_copy(idx_hbm.at[pl.ds(wid * ROWS_PER_TILE, ROWS_PER_TILE)],
                    idx_v)
    plsc.subcore_barrier()

    def gathers(grp, rows_buf, sem, fire):
      for j in range(K):
        src = p_hbm.at[idx_v.at[grp * K + j, 0]]
        dst = rows_buf.at[pl.ds(j * ROW, ROW)]
        if fire:
          pltpu.async_copy(src, dst, sem)
        else:
          pltpu.make_async_copy(src, dst, sem).wait()

    def scatters(grp, rows_buf, sem, fire):
      for j in range(K):
        src = rows_buf.at[pl.ds(j * ROW, ROW)]
        dst = acc_sh.at[idx_v.at[grp * K + j, 1]]
        if fire:
          pltpu.async_copy(src, dst, sem, add=True)
        else:
          pltpu.make_async_copy(src, dst, sem).wait()

    FIRE, DRAIN = True, False

    gathers(0, rows_a, gsem_a, FIRE)

    @pl.loop(0, PAIRS)
    def _(i):
      ge = 2 * i
      gathers(ge + 1, rows_b, gsem_b, FIRE)
      gathers(ge, rows_a, gsem_a, DRAIN)
      scatters(ge, rows_a, ssem_a, FIRE)
      scatters(ge, rows_a, ssem_a, DRAIN)

      @pl.when(i < PAIRS - 1)
      def _():
        gathers(ge + 2, rows_a, gsem_a, FIRE)

      gathers(ge + 1, rows_b, gsem_b, DRAIN)
      scatters(ge + 1, rows_b, ssem_b, FIRE)
      scatters(ge + 1, rows_b, ssem_b, DRAIN)

    plsc.subcore_barrier()

    nr = s * N_PER_TILE
    pltpu.sync_copy(acc_sh.at[pl.ds(nr, N_PER_TILE)],
                    acc_out.at[c, pl.ds(nr, N_PER_TILE)])

  mesh = plsc.VectorSubcoreMesh(core_axis_name="c", subcore_axis_name="s",
                                num_cores=NC, num_subcores=NS)
  return pl.kernel(
      body, out_type=out_type, mesh=mesh, scratch_types=scratch,
      compiler_params=pltpu.CompilerParams(use_tc_tiling_on_sc=False))


def _edge_pass_aug(p, idx):
  return _make_edge_pass(D_AUG)(p, idx)


def _edge_pass(p, idx):
  return _make_edge_pass(D_H)(p, idx)


def _dot(a, b):
  return lax.dot_general(a, b, (((1,), (0,)), ((), ())),
                         precision=lax.Precision.HIGHEST,
                         preferred_element_type=jnp.float32)


def _proj2_aug_body(x_ref, wl_ref, wr_ref, p_ref, r_ref):
  x = x_ref[...]
  p_ref[:, :D_H] = _dot(x, wl_ref[...])
  p_ref[:, D_H:] = jnp.ones((N_NODES, D_AUG - D_H), jnp.float32)
  r_ref[...] = _dot(x, wr_ref[...])


def _proj2_aug(x, wl, wr):
  return pl.pallas_call(
      _proj2_aug_body,
      out_shape=(jax.ShapeDtypeStruct((N_NODES, D_AUG), jnp.float32),
                 jax.ShapeDtypeStruct((N_NODES, D_H), jnp.float32)),
  )(x, wl, wr)


def _proj2_body(x_ref, wl_ref, wr_ref, p_ref, r_ref):
  x = x_ref[...]
  p_ref[...] = _dot(x, wl_ref[...])
  r_ref[...] = _dot(x, wr_ref[...])


def _proj2(x, wl, wr):
  return pl.pallas_call(
      _proj2_body,
      out_shape=(jax.ShapeDtypeStruct((N_NODES, D_H), jnp.float32),
                 jax.ShapeDtypeStruct((N_NODES, D_H), jnp.float32)),
  )(x, wl, wr)


def _bn_relu(h, gamma, beta):
  mu = jnp.mean(h, axis=0, keepdims=True)
  var = jnp.mean(jnp.square(h - mu), axis=0, keepdims=True)
  hn = (h - mu) * lax.rsqrt(var + 1e-5) * gamma + beta
  return jnp.maximum(hn, 0.0)


def _bn0_body(acc_ref, r_ref, b_ref, g_ref, be_ref, h_ref, dinv_ref):
  agg = acc_ref[0, :N_NODES, :D_H] + acc_ref[1, :N_NODES, :D_H]
  deg = acc_ref[0, :N_NODES, D_H:D_H + 1] + acc_ref[1, :N_NODES, D_H:D_H + 1]
  dinv = 1.0 / jnp.maximum(deg, 1.0)
  dinv_ref[...] = dinv
  h = agg * dinv + b_ref[...] + r_ref[...]
  h_ref[...] = _bn_relu(h, g_ref[...], be_ref[...])


def _bn0(acc, r, b, gamma, beta):
  return pl.pallas_call(
      _bn0_body,
      out_shape=(jax.ShapeDtypeStruct((N_NODES, D_H), jnp.float32),
                 jax.ShapeDtypeStruct((N_NODES, 1), jnp.float32)),
  )(acc, r, b, gamma, beta)


def _bn1_body(acc_ref, dinv_ref, r_ref, b_ref, g_ref, be_ref, h_ref):
  agg = acc_ref[0, :N_NODES, :D_H] + acc_ref[1, :N_NODES, :D_H]
  h = agg * dinv_ref[...] + b_ref[...] + r_ref[...]
  h_ref[...] = _bn_relu(h, g_ref[...], be_ref[...])


def _bn1(acc, dinv, r, b, gamma, beta):
  return pl.pallas_call(
      _bn1_body,
      out_shape=jax.ShapeDtypeStruct((N_NODES, D_H), jnp.float32),
  )(acc, dinv, r, b, gamma, beta)


def _score_body(h_ref, ded_ref, bw_ref, bb_ref, out_ref):
  hw = _dot(h_ref[...], bw_ref[...])
  scores = lax.dot_general(hw, ded_ref[...], (((1,), (1,)), ((), ())),
                           precision=lax.Precision.HIGHEST,
                           preferred_element_type=jnp.float32)
  out_ref[...] = jax.nn.sigmoid(scores + bb_ref[0, 0])


def _score(h2, ded_table, bil_W, bil_b):
  return pl.pallas_call(
      _score_body,
      out_shape=jax.ShapeDtypeStruct((N_NODES, N_DED_OUT), jnp.float32),
  )(h2, ded_table, bil_W, bil_b)


def kernel(user_x, edge_index, W_l0, b_l0, W_r0, W_l1, b_l1, W_r1,
           gamma0, beta0, gamma1, beta1, ded_table, bil_W, bil_b):
  npad = E_PAD - N_EDGES
  src_p = jnp.concatenate([edge_index[0], jnp.zeros((npad,), jnp.int32)])
  dst_p = jnp.concatenate([edge_index[1], jnp.full((npad,), N_NODES, jnp.int32)])
  idx = jnp.stack([src_p.reshape(E_PAD // ROW, ROW),
                   dst_p.reshape(E_PAD // ROW, ROW)], axis=1)
  b_l0_ = b_l0.reshape(1, D_H)
  b_l1_ = b_l1.reshape(1, D_H)
  gamma0_ = gamma0.reshape(1, D_H)
  beta0_ = beta0.reshape(1, D_H)
  gamma1_ = gamma1.reshape(1, D_H)
  beta1_ = beta1.reshape(1, D_H)
  bil_b_ = bil_b.reshape(1, 1)

  p0, r0 = _proj2_aug(user_x, W_l0, W_r0)
  acc0 = _edge_pass_aug(p0, idx)
  h1, dinv = _bn0(acc0, r0, b_l0_, gamma0_, beta0_)
  p1, r1 = _proj2_aug(h1, W_l1, W_r1)
  acc1 = _edge_pass_aug(p1, idx)
  h2 = _bn1(acc1, dinv, r1, b_l1_, gamma1_, beta1_)
  return _score(h2, ded_table, bil_W, bil_b_)


# speedup vs baseline: 13.3384x; 2.6653x over previous
import functools

import jax
import jax.numpy as jnp
from jax import lax
from jax.experimental import pallas as pl
from jax.experimental.pallas import tpu as pltpu
from jax.experimental.pallas import tpu_sc as plsc

N_NODES = 10000
D_IN = 128
D_H = 64
N_EDGES = 320000
N_DED_OUT = 8

NC = 2
NS = 16
NW = NC * NS
ROW = 128
K = 2
ROWS_PER_TILE = 80
GROUPS = ROWS_PER_TILE // K
PAIRS = GROUPS // 2
E_ROWS = N_EDGES // ROW
LAST_ROWS = E_ROWS - (NW - 1) * ROWS_PER_TILE
N_PAD = 10240
N_PER_TILE = N_PAD // NS
D_AUG = D_H + 16


@functools.lru_cache(maxsize=None)
def _make_edge_pass(width: int):
  out_type = jax.ShapeDtypeStruct((NC, N_PAD, width), jnp.float32)
  scratch = [
      pltpu.VMEM_SHARED((N_PAD, width), jnp.float32),
      pltpu.VMEM((ROWS_PER_TILE, ROW), jnp.int32),
      pltpu.VMEM((ROWS_PER_TILE, ROW), jnp.int32),
      pltpu.VMEM((K * ROW, width), jnp.float32),
      pltpu.VMEM((K * ROW, width), jnp.float32),
      pltpu.VMEM((ROW, width), jnp.float32),
      pltpu.SemaphoreType.DMA,
      pltpu.SemaphoreType.DMA,
      pltpu.SemaphoreType.DMA,
      pltpu.SemaphoreType.DMA,
  ]

  def body(p_hbm, e_hbm, acc_out, acc_sh, src_v, dst_v,
           rows_a, rows_b, zbuf_v, gsem_a, gsem_b, ssem_a, ssem_b):
    c = lax.axis_index("c")
    s = lax.axis_index("s")

    @pl.loop(0, ROW)
    def _(r):
      for j in range(width // 16):
        zbuf_v[r, pl.ds(j * 16, 16)] = jnp.zeros((16,), jnp.float32)

    for k in range(N_PER_TILE // ROW):
      pltpu.sync_copy(zbuf_v, acc_sh.at[pl.ds(s * N_PER_TILE + k * ROW, ROW)])

    wid = s * NC + c
    r0 = wid * ROWS_PER_TILE
    pltpu.sync_copy(e_hbm.at[0, pl.ds(r0, LAST_ROWS)],
                    src_v.at[pl.ds(0, LAST_ROWS)])
    pltpu.sync_copy(e_hbm.at[1, pl.ds(r0, LAST_ROWS)],
                    dst_v.at[pl.ds(0, LAST_ROWS)])

    @pl.when(wid < NW - 1)
    def _():
      pltpu.sync_copy(e_hbm.at[0, pl.ds(r0 + LAST_ROWS, ROWS_PER_TILE - LAST_ROWS)],
                      src_v.at[pl.ds(LAST_ROWS, ROWS_PER_TILE - LAST_ROWS)])
      pltpu.sync_copy(e_hbm.at[1, pl.ds(r0 + LAST_ROWS, ROWS_PER_TILE - LAST_ROWS)],
                      dst_v.at[pl.ds(LAST_ROWS, ROWS_PER_TILE - LAST_ROWS)])
    plsc.subcore_barrier()
    pairs_t = jnp.where(wid == NW - 1, LAST_ROWS // K // 2, PAIRS)

    def gathers(grp, rows_buf, sem, fire):
      for j in range(K):
        src = p_hbm.at[src_v.at[grp * K + j]]
        dst = rows_buf.at[pl.ds(j * ROW, ROW)]
        if fire:
          pltpu.async_copy(src, dst, sem)
        else:
          pltpu.make_async_copy(src, dst, sem).wait()

    def scatters(grp, rows_buf, sem, fire):
      for j in range(K):
        src = rows_buf.at[pl.ds(j * ROW, ROW)]
        dst = acc_sh.at[dst_v.at[grp * K + j]]
        if fire:
          pltpu.async_copy(src, dst, sem, add=True)
        else:
          pltpu.make_async_copy(src, dst, sem).wait()

    FIRE, DRAIN = True, False

    gathers(0, rows_a, gsem_a, FIRE)

    @pl.loop(0, pairs_t)
    def _(i):
      ge = 2 * i
      gathers(ge + 1, rows_b, gsem_b, FIRE)
      gathers(ge, rows_a, gsem_a, DRAIN)
      scatters(ge, rows_a, ssem_a, FIRE)
      scatters(ge, rows_a, ssem_a, DRAIN)

      @pl.when(i < pairs_t - 1)
      def _():
        gathers(ge + 2, rows_a, gsem_a, FIRE)

      gathers(ge + 1, rows_b, gsem_b, DRAIN)
      scatters(ge + 1, rows_b, ssem_b, FIRE)
      scatters(ge + 1, rows_b, ssem_b, DRAIN)

    plsc.subcore_barrier()

    nr = s * N_PER_TILE
    pltpu.sync_copy(acc_sh.at[pl.ds(nr, N_PER_TILE)],
                    acc_out.at[c, pl.ds(nr, N_PER_TILE)])

  mesh = plsc.VectorSubcoreMesh(core_axis_name="c", subcore_axis_name="s",
                                num_cores=NC, num_subcores=NS)
  return pl.kernel(
      body, out_type=out_type, mesh=mesh, scratch_types=scratch,
      compiler_params=pltpu.CompilerParams(use_tc_tiling_on_sc=False))


def _edge_pass_aug(p, e3):
  return _make_edge_pass(D_AUG)(p, e3)


def _dot(a, b):
  return lax.dot_general(a, b, (((1,), (0,)), ((), ())),
                         precision=lax.Precision.HIGHEST,
                         preferred_element_type=jnp.float32)


def _proj2_aug_body(x_ref, wl_ref, wr_ref, p_ref, r_ref):
  x = x_ref[...]
  p_ref[:, :D_H] = _dot(x, wl_ref[...])
  p_ref[:, D_H:] = jnp.ones((N_NODES, D_AUG - D_H), jnp.float32)
  r_ref[...] = _dot(x, wr_ref[...])


def _proj2_aug(x, wl, wr):
  return pl.pallas_call(
      _proj2_aug_body,
      out_shape=(jax.ShapeDtypeStruct((N_NODES, D_AUG), jnp.float32),
                 jax.ShapeDtypeStruct((N_NODES, D_H), jnp.float32)),
  )(x, wl, wr)


def _proj2_body(x_ref, wl_ref, wr_ref, p_ref, r_ref):
  x = x_ref[...]
  p_ref[...] = _dot(x, wl_ref[...])
  r_ref[...] = _dot(x, wr_ref[...])


def _proj2(x, wl, wr):
  return pl.pallas_call(
      _proj2_body,
      out_shape=(jax.ShapeDtypeStruct((N_NODES, D_H), jnp.float32),
                 jax.ShapeDtypeStruct((N_NODES, D_H), jnp.float32)),
  )(x, wl, wr)


def _bn_relu(h, gamma, beta):
  mu = jnp.mean(h, axis=0, keepdims=True)
  var = jnp.mean(jnp.square(h - mu), axis=0, keepdims=True)
  hn = (h - mu) * lax.rsqrt(var + 1e-5) * gamma + beta
  return jnp.maximum(hn, 0.0)


def _bn0_body(acc_ref, r_ref, b_ref, g_ref, be_ref, h_ref, dinv_ref):
  agg = acc_ref[0, :N_NODES, :D_H] + acc_ref[1, :N_NODES, :D_H]
  deg = acc_ref[0, :N_NODES, D_H:D_H + 1] + acc_ref[1, :N_NODES, D_H:D_H + 1]
  dinv = 1.0 / jnp.maximum(deg, 1.0)
  dinv_ref[...] = dinv
  h = agg * dinv + b_ref[...] + r_ref[...]
  h_ref[...] = _bn_relu(h, g_ref[...], be_ref[...])


def _bn0(acc, r, b, gamma, beta):
  return pl.pallas_call(
      _bn0_body,
      out_shape=(jax.ShapeDtypeStruct((N_NODES, D_H), jnp.float32),
                 jax.ShapeDtypeStruct((N_NODES, 1), jnp.float32)),
  )(acc, r, b, gamma, beta)


def _bn1_body(acc_ref, dinv_ref, r_ref, b_ref, g_ref, be_ref, h_ref):
  agg = acc_ref[0, :N_NODES, :D_H] + acc_ref[1, :N_NODES, :D_H]
  h = agg * dinv_ref[...] + b_ref[...] + r_ref[...]
  h_ref[...] = _bn_relu(h, g_ref[...], be_ref[...])


def _bn1(acc, dinv, r, b, gamma, beta):
  return pl.pallas_call(
      _bn1_body,
      out_shape=jax.ShapeDtypeStruct((N_NODES, D_H), jnp.float32),
  )(acc, dinv, r, b, gamma, beta)


def _score_body(h_ref, ded_ref, bw_ref, bb_ref, out_ref):
  hw = _dot(h_ref[...], bw_ref[...])
  scores = lax.dot_general(hw, ded_ref[...], (((1,), (1,)), ((), ())),
                           precision=lax.Precision.HIGHEST,
                           preferred_element_type=jnp.float32)
  out_ref[...] = jax.nn.sigmoid(scores + bb_ref[0, 0])


def _score(h2, ded_table, bil_W, bil_b):
  return pl.pallas_call(
      _score_body,
      out_shape=jax.ShapeDtypeStruct((N_NODES, N_DED_OUT), jnp.float32),
  )(h2, ded_table, bil_W, bil_b)


def kernel(user_x, edge_index, W_l0, b_l0, W_r0, W_l1, b_l1, W_r1,
           gamma0, beta0, gamma1, beta1, ded_table, bil_W, bil_b):
  e3 = edge_index.reshape(2, E_ROWS, ROW)
  b_l0_ = b_l0.reshape(1, D_H)
  b_l1_ = b_l1.reshape(1, D_H)
  gamma0_ = gamma0.reshape(1, D_H)
  beta0_ = beta0.reshape(1, D_H)
  gamma1_ = gamma1.reshape(1, D_H)
  beta1_ = beta1.reshape(1, D_H)
  bil_b_ = bil_b.reshape(1, 1)

  p0, r0 = _proj2_aug(user_x, W_l0, W_r0)
  acc0 = _edge_pass_aug(p0, e3)
  h1, dinv = _bn0(acc0, r0, b_l0_, gamma0_, beta0_)
  p1, r1 = _proj2_aug(h1, W_l1, W_r1)
  acc1 = _edge_pass_aug(p1, e3)
  h2 = _bn1(acc1, dinv, r1, b_l1_, gamma1_, beta1_)
  return _score(h2, ded_table, bil_W, bil_b_)


# speedup vs baseline: 14.4057x; 1.0800x over previous
import functools

import jax
import jax.numpy as jnp
from jax import lax
from jax.experimental import pallas as pl
from jax.experimental.pallas import tpu as pltpu
from jax.experimental.pallas import tpu_sc as plsc

N_NODES = 10000
D_IN = 128
D_H = 64
N_EDGES = 320000
N_DED_OUT = 8

NC = 2
NS = 16
NW = NC * NS
ROW = 128
K = 2
ROWS_PER_TILE = 80
GROUPS = ROWS_PER_TILE // K
PAIRS = GROUPS // 2
E_ROWS = N_EDGES // ROW
LAST_ROWS = E_ROWS - (NW - 1) * ROWS_PER_TILE
N_PAD = 10240
N_PER_TILE = N_PAD // NS
D_AUG = D_H + 16


@functools.lru_cache(maxsize=None)
def _make_edge_pass(width: int):
  out_type = jax.ShapeDtypeStruct((NC, N_PAD, width), jnp.float32)
  scratch = [
      pltpu.VMEM_SHARED((N_PAD, width), jnp.float32),
      pltpu.VMEM((ROWS_PER_TILE, ROW), jnp.int32),
      pltpu.VMEM((ROWS_PER_TILE, ROW), jnp.int32),
      pltpu.VMEM((K * ROW, width), jnp.float32),
      pltpu.VMEM((K * ROW, width), jnp.float32),
      pltpu.VMEM((ROW, width), jnp.float32),
      pltpu.SemaphoreType.DMA,
      pltpu.SemaphoreType.DMA,
      pltpu.SemaphoreType.DMA,
      pltpu.SemaphoreType.DMA,
  ]

  def body(p_hbm, e_hbm, acc_out, acc_sh, src_v, dst_v,
           rows_a, rows_b, zbuf_v, gsem_a, gsem_b, ssem_a, ssem_b):
    c = lax.axis_index("c")
    s = lax.axis_index("s")

    wid = s * NC + c
    r0 = wid * ROWS_PER_TILE
    pltpu.sync_copy(e_hbm.at[0, pl.ds(r0, LAST_ROWS)],
                    src_v.at[pl.ds(0, LAST_ROWS)])
    pltpu.sync_copy(e_hbm.at[1, pl.ds(r0, LAST_ROWS)],
                    dst_v.at[pl.ds(0, LAST_ROWS)])

    @pl.when(wid < NW - 1)
    def _():
      pltpu.sync_copy(e_hbm.at[0, pl.ds(r0 + LAST_ROWS, ROWS_PER_TILE - LAST_ROWS)],
                      src_v.at[pl.ds(LAST_ROWS, ROWS_PER_TILE - LAST_ROWS)])
      pltpu.sync_copy(e_hbm.at[1, pl.ds(r0 + LAST_ROWS, ROWS_PER_TILE - LAST_ROWS)],
                      dst_v.at[pl.ds(LAST_ROWS, ROWS_PER_TILE - LAST_ROWS)])
    pairs_t = jnp.where(wid == NW - 1, LAST_ROWS // K // 2, PAIRS)

    def gathers(grp, rows_buf, sem, fire):
      for j in range(K):
        src = p_hbm.at[src_v.at[grp * K + j]]
        dst = rows_buf.at[pl.ds(j * ROW, ROW)]
        if fire:
          pltpu.async_copy(src, dst, sem)
        else:
          pltpu.make_async_copy(src, dst, sem).wait()

    def scatters(grp, rows_buf, sem, fire):
      for j in range(K):
        src = rows_buf.at[pl.ds(j * ROW, ROW)]
        dst = acc_sh.at[dst_v.at[grp * K + j]]
        if fire:
          pltpu.async_copy(src, dst, sem, add=True)
        else:
          pltpu.make_async_copy(src, dst, sem).wait()

    FIRE, DRAIN = True, False

    gathers(0, rows_a, gsem_a, FIRE)
    gathers(1, rows_b, gsem_b, FIRE)

    @pl.loop(0, ROW)
    def _(r):
      for j in range(width // 16):
        zbuf_v[r, pl.ds(j * 16, 16)] = jnp.zeros((16,), jnp.float32)

    for k in range(N_PER_TILE // ROW):
      pltpu.sync_copy(zbuf_v, acc_sh.at[pl.ds(s * N_PER_TILE + k * ROW, ROW)])
    plsc.subcore_barrier()

    @pl.loop(0, pairs_t)
    def _(i):
      ge = 2 * i

      @pl.when(i > 0)
      def _():
        gathers(ge + 1, rows_b, gsem_b, FIRE)
      gathers(ge, rows_a, gsem_a, DRAIN)
      scatters(ge, rows_a, ssem_a, FIRE)
      scatters(ge, rows_a, ssem_a, DRAIN)

      @pl.when(i < pairs_t - 1)
      def _():
        gathers(ge + 2, rows_a, gsem_a, FIRE)

      gathers(ge + 1, rows_b, gsem_b, DRAIN)
      scatters(ge + 1, rows_b, ssem_b, FIRE)
      scatters(ge + 1, rows_b, ssem_b, DRAIN)

    plsc.subcore_barrier()

    nr = s * N_PER_TILE
    pltpu.sync_copy(acc_sh.at[pl.ds(nr, N_PER_TILE)],
                    acc_out.at[c, pl.ds(nr, N_PER_TILE)])

  mesh = plsc.VectorSubcoreMesh(core_axis_name="c", subcore_axis_name="s",
                                num_cores=NC, num_subcores=NS)
  return pl.kernel(
      body, out_type=out_type, mesh=mesh, scratch_types=scratch,
      compiler_params=pltpu.CompilerParams(use_tc_tiling_on_sc=False))


def _edge_pass_aug(p, e3):
  return _make_edge_pass(D_AUG)(p, e3)


def _edge_pass(p, e3):
  return _make_edge_pass(D_H)(p, e3)


def _dot(a, b):
  return lax.dot_general(a, b, (((1,), (0,)), ((), ())),
                         precision=lax.Precision.HIGHEST,
                         preferred_element_type=jnp.float32)


def _proj2_aug_body(x_ref, wl_ref, wr_ref, p_ref, r_ref):
  x = x_ref[...]
  p_ref[:, :D_H] = _dot(x, wl_ref[...])
  p_ref[:, D_H:] = jnp.ones((N_NODES, D_AUG - D_H), jnp.float32)
  r_ref[...] = _dot(x, wr_ref[...])


def _proj2_aug(x, wl, wr):
  return pl.pallas_call(
      _proj2_aug_body,
      out_shape=(jax.ShapeDtypeStruct((N_NODES, D_AUG), jnp.float32),
                 jax.ShapeDtypeStruct((N_NODES, D_H), jnp.float32)),
  )(x, wl, wr)


def _proj2_body(x_ref, wl_ref, wr_ref, p_ref, r_ref):
  x = x_ref[...]
  p_ref[...] = _dot(x, wl_ref[...])
  r_ref[...] = _dot(x, wr_ref[...])


def _proj2(x, wl, wr):
  return pl.pallas_call(
      _proj2_body,
      out_shape=(jax.ShapeDtypeStruct((N_NODES, D_H), jnp.float32),
                 jax.ShapeDtypeStruct((N_NODES, D_H), jnp.float32)),
  )(x, wl, wr)


def _bn_relu(h, gamma, beta):
  mu = jnp.mean(h, axis=0, keepdims=True)
  var = jnp.mean(jnp.square(h - mu), axis=0, keepdims=True)
  hn = (h - mu) * lax.rsqrt(var + 1e-5) * gamma + beta
  return jnp.maximum(hn, 0.0)


def _bn0_body(acc_ref, r_ref, b_ref, g_ref, be_ref, h_ref, dinv_ref):
  agg = acc_ref[0, :N_NODES, :D_H] + acc_ref[1, :N_NODES, :D_H]
  deg = acc_ref[0, :N_NODES, D_H:D_H + 1] + acc_ref[1, :N_NODES, D_H:D_H + 1]
  dinv = 1.0 / jnp.maximum(deg, 1.0)
  dinv_ref[...] = dinv
  h = agg * dinv + b_ref[...] + r_ref[...]
  h_ref[...] = _bn_relu(h, g_ref[...], be_ref[...])


def _bn0(acc, r, b, gamma, beta):
  return pl.pallas_call(
      _bn0_body,
      out_shape=(jax.ShapeDtypeStruct((N_NODES, D_H), jnp.float32),
                 jax.ShapeDtypeStruct((N_NODES, 1), jnp.float32)),
  )(acc, r, b, gamma, beta)


def _bn1_body(acc_ref, dinv_ref, r_ref, b_ref, g_ref, be_ref, h_ref):
  agg = acc_ref[0, :N_NODES, :D_H] + acc_ref[1, :N_NODES, :D_H]
  h = agg * dinv_ref[...] + b_ref[...] + r_ref[...]
  h_ref[...] = _bn_relu(h, g_ref[...], be_ref[...])


def _bn1(acc, dinv, r, b, gamma, beta):
  return pl.pallas_call(
      _bn1_body,
      out_shape=jax.ShapeDtypeStruct((N_NODES, D_H), jnp.float32),
  )(acc, dinv, r, b, gamma, beta)


def _score_body(h_ref, ded_ref, bw_ref, bb_ref, out_ref):
  hw = _dot(h_ref[...], bw_ref[...])
  scores = lax.dot_general(hw, ded_ref[...], (((1,), (1,)), ((), ())),
                           precision=lax.Precision.HIGHEST,
                           preferred_element_type=jnp.float32)
  out_ref[...] = jax.nn.sigmoid(scores + bb_ref[0, 0])


def _score(h2, ded_table, bil_W, bil_b):
  return pl.pallas_call(
      _score_body,
      out_shape=jax.ShapeDtypeStruct((N_NODES, N_DED_OUT), jnp.float32),
  )(h2, ded_table, bil_W, bil_b)


def kernel(user_x, edge_index, W_l0, b_l0, W_r0, W_l1, b_l1, W_r1,
           gamma0, beta0, gamma1, beta1, ded_table, bil_W, bil_b):
  e3 = edge_index.reshape(2, E_ROWS, ROW)
  b_l0_ = b_l0.reshape(1, D_H)
  b_l1_ = b_l1.reshape(1, D_H)
  gamma0_ = gamma0.reshape(1, D_H)
  beta0_ = beta0.reshape(1, D_H)
  gamma1_ = gamma1.reshape(1, D_H)
  beta1_ = beta1.reshape(1, D_H)
  bil_b_ = bil_b.reshape(1, 1)

  p0, r0 = _proj2_aug(user_x, W_l0, W_r0)
  acc0 = _edge_pass_aug(p0, e3)
  h1, dinv = _bn0(acc0, r0, b_l0_, gamma0_, beta0_)
  p1, r1 = _proj2(h1, W_l1, W_r1)
  acc1 = _edge_pass(p1, e3)
  h2 = _bn1(acc1, dinv, r1, b_l1_, gamma1_, beta1_)
  return _score(h2, ded_table, bil_W, bil_b_)


# speedup vs baseline: 15.6587x; 1.0870x over previous
import functools

import jax
import jax.numpy as jnp
from jax import lax
from jax.experimental import pallas as pl
from jax.experimental.pallas import tpu as pltpu
from jax.experimental.pallas import tpu_sc as plsc

N_NODES = 10000
D_IN = 128
D_H = 64
N_EDGES = 320000
N_DED_OUT = 8

NC = 2
NS = 16
NW = NC * NS
ROW = 128
K = 2
ROWS_PER_TILE = 80
GROUPS = ROWS_PER_TILE // K
PAIRS = GROUPS // 2
E_ROWS = N_EDGES // ROW
LAST_ROWS = E_ROWS - (NW - 1) * ROWS_PER_TILE
N_PAD = 10240
N_PER_TILE = N_PAD // NS
D_AUG = D_H + 16


@functools.lru_cache(maxsize=None)
def _make_edge_pass(width: int):
  out_type = jax.ShapeDtypeStruct((NC, N_PAD, width), jnp.float32)
  scratch = [
      pltpu.VMEM_SHARED((N_PAD, width), jnp.float32),
      pltpu.VMEM((ROWS_PER_TILE, ROW), jnp.int32),
      pltpu.VMEM((ROWS_PER_TILE, ROW), jnp.int32),
      pltpu.VMEM((K * ROW, width), jnp.float32),
      pltpu.VMEM((K * ROW, width), jnp.float32),
      pltpu.VMEM((ROW, width), jnp.float32),
      pltpu.SemaphoreType.DMA,
      pltpu.SemaphoreType.DMA,
      pltpu.SemaphoreType.DMA,
      pltpu.SemaphoreType.DMA,
  ]

  def body(p_hbm, e_hbm, acc_out, acc_sh, src_v, dst_v,
           rows_a, rows_b, zbuf_v, gsem_a, gsem_b, ssem_a, ssem_b):
    c = lax.axis_index("c")
    s = lax.axis_index("s")

    wid = s * NC + c
    r0 = wid * ROWS_PER_TILE
    pltpu.sync_copy(e_hbm.at[0, pl.ds(r0, LAST_ROWS)],
                    src_v.at[pl.ds(0, LAST_ROWS)])
    pltpu.sync_copy(e_hbm.at[1, pl.ds(r0, LAST_ROWS)],
                    dst_v.at[pl.ds(0, LAST_ROWS)])

    @pl.when(wid < NW - 1)
    def _():
      pltpu.sync_copy(e_hbm.at[0, pl.ds(r0 + LAST_ROWS, ROWS_PER_TILE - LAST_ROWS)],
                      src_v.at[pl.ds(LAST_ROWS, ROWS_PER_TILE - LAST_ROWS)])
      pltpu.sync_copy(e_hbm.at[1, pl.ds(r0 + LAST_ROWS, ROWS_PER_TILE - LAST_ROWS)],
                      dst_v.at[pl.ds(LAST_ROWS, ROWS_PER_TILE - LAST_ROWS)])
    pairs_t = jnp.where(wid == NW - 1, LAST_ROWS // K // 2, PAIRS)

    def gathers(grp, rows_buf, sem, fire):
      for j in range(K):
        src = p_hbm.at[src_v.at[grp * K + j]]
        dst = rows_buf.at[pl.ds(j * ROW, ROW)]
        if fire:
          pltpu.async_copy(src, dst, sem)
        else:
          pltpu.make_async_copy(src, dst, sem).wait()

    def scatters(grp, rows_buf, sem, fire):
      for j in range(K):
        src = rows_buf.at[pl.ds(j * ROW, ROW)]
        dst = acc_sh.at[dst_v.at[grp * K + j]]
        if fire:
          pltpu.async_copy(src, dst, sem, add=True)
        else:
          pltpu.make_async_copy(src, dst, sem).wait()

    FIRE, DRAIN = True, False

    gathers(0, rows_a, gsem_a, FIRE)
    gathers(1, rows_b, gsem_b, FIRE)

    @pl.loop(0, ROW)
    def _(r):
      for j in range(width // 16):
        zbuf_v[r, pl.ds(j * 16, 16)] = jnp.zeros((16,), jnp.float32)

    for k in range(N_PER_TILE // ROW):
      pltpu.sync_copy(zbuf_v, acc_sh.at[pl.ds(s * N_PER_TILE + k * ROW, ROW)])
    plsc.subcore_barrier()

    @pl.loop(0, pairs_t)
    def _(i):
      ge = 2 * i

      @pl.when(i > 0)
      def _():
        gathers(ge + 1, rows_b, gsem_b, FIRE)
      gathers(ge, rows_a, gsem_a, DRAIN)
      scatters(ge, rows_a, ssem_a, FIRE)
      scatters(ge, rows_a, ssem_a, DRAIN)

      @pl.when(i < pairs_t - 1)
      def _():
        gathers(ge + 2, rows_a, gsem_a, FIRE)

      gathers(ge + 1, rows_b, gsem_b, DRAIN)
      scatters(ge + 1, rows_b, ssem_b, FIRE)
      scatters(ge + 1, rows_b, ssem_b, DRAIN)

    plsc.subcore_barrier()

    nr = s * N_PER_TILE
    pltpu.sync_copy(acc_sh.at[pl.ds(nr, N_PER_TILE)],
                    acc_out.at[c, pl.ds(nr, N_PER_TILE)])

  mesh = plsc.VectorSubcoreMesh(core_axis_name="c", subcore_axis_name="s",
                                num_cores=NC, num_subcores=NS)
  return pl.kernel(
      body, out_type=out_type, mesh=mesh, scratch_types=scratch,
      compiler_params=pltpu.CompilerParams(use_tc_tiling_on_sc=False))


def _edge_pass_aug(p, e3):
  return _make_edge_pass(D_AUG)(p, e3)


def _edge_pass(p, e3):
  return _make_edge_pass(D_H)(p, e3)


def _dot(a, b):
  return lax.dot_general(a, b, (((1,), (0,)), ((), ())),
                         precision=lax.Precision.HIGHEST,
                         preferred_element_type=jnp.float32)


def _proj2_aug_body(x_ref, wl_ref, wr_ref, p_ref, r_ref):
  x = x_ref[...]
  p_ref[:, :D_H] = _dot(x, wl_ref[...])
  p_ref[:, D_H:] = jnp.ones((N_NODES, D_AUG - D_H), jnp.float32)
  r_ref[...] = _dot(x, wr_ref[...])


def _proj2_aug(x, wl, wr):
  return pl.pallas_call(
      _proj2_aug_body,
      out_shape=(jax.ShapeDtypeStruct((N_NODES, D_AUG), jnp.float32),
                 jax.ShapeDtypeStruct((N_NODES, D_H), jnp.float32)),
  )(x, wl, wr)


def _proj2_body(x_ref, wl_ref, wr_ref, p_ref, r_ref):
  x = x_ref[...]
  p_ref[...] = _dot(x, wl_ref[...])
  r_ref[...] = _dot(x, wr_ref[...])


def _proj2(x, wl, wr):
  return pl.pallas_call(
      _proj2_body,
      out_shape=(jax.ShapeDtypeStruct((N_NODES, D_H), jnp.float32),
                 jax.ShapeDtypeStruct((N_NODES, D_H), jnp.float32)),
  )(x, wl, wr)


def _bn_relu(h, gamma, beta):
  n = jnp.float32(h.shape[0])
  mu = jnp.sum(h, axis=0, keepdims=True) / n
  var = jnp.sum(jnp.square(h), axis=0, keepdims=True) / n - jnp.square(mu)
  scale = lax.rsqrt(var + 1e-5) * gamma
  return jnp.maximum(h * scale + (beta - mu * scale), 0.0)


def _bn0_body(acc_ref, r_ref, b_ref, g_ref, be_ref, h_ref, dinv_ref):
  agg = acc_ref[0, :N_NODES, :D_H] + acc_ref[1, :N_NODES, :D_H]
  deg = acc_ref[0, :N_NODES, D_H:D_H + 1] + acc_ref[1, :N_NODES, D_H:D_H + 1]
  dinv = 1.0 / jnp.maximum(deg, 1.0)
  dinv_ref[...] = dinv
  h = agg * dinv + b_ref[...] + r_ref[...]
  h_ref[...] = _bn_relu(h, g_ref[...], be_ref[...])


def _bn0(acc, r, b, gamma, beta):
  return pl.pallas_call(
      _bn0_body,
      out_shape=(jax.ShapeDtypeStruct((N_NODES, D_H), jnp.float32),
                 jax.ShapeDtypeStruct((N_NODES, 1), jnp.float32)),
  )(acc, r, b, gamma, beta)


def _final_body(acc_ref, dinv_ref, r_ref, b_ref, g_ref, be_ref, ded_ref,
                bw_ref, bb_ref, out_ref):
  agg = acc_ref[0, :N_NODES, :D_H] + acc_ref[1, :N_NODES, :D_H]
  h = agg * dinv_ref[...] + b_ref[...] + r_ref[...]
  h2 = _bn_relu(h, g_ref[...], be_ref[...])
  m = lax.dot_general(bw_ref[...], ded_ref[...], (((1,), (1,)), ((), ())),
                      precision=lax.Precision.HIGHEST,
                      preferred_element_type=jnp.float32)
  scores = _dot(h2, m)
  out_ref[...] = jax.nn.sigmoid(scores + bb_ref[0, 0])


def _final(acc, dinv, r, b, gamma, beta, ded_table, bil_W, bil_b):
  return pl.pallas_call(
      _final_body,
      out_shape=jax.ShapeDtypeStruct((N_NODES, N_DED_OUT), jnp.float32),
  )(acc, dinv, r, b, gamma, beta, ded_table, bil_W, bil_b)


def kernel(user_x, edge_index, W_l0, b_l0, W_r0, W_l1, b_l1, W_r1,
           gamma0, beta0, gamma1, beta1, ded_table, bil_W, bil_b):
  e3 = edge_index.reshape(2, E_ROWS, ROW)
  b_l0_ = b_l0.reshape(1, D_H)
  b_l1_ = b_l1.reshape(1, D_H)
  gamma0_ = gamma0.reshape(1, D_H)
  beta0_ = beta0.reshape(1, D_H)
  gamma1_ = gamma1.reshape(1, D_H)
  beta1_ = beta1.reshape(1, D_H)
  bil_b_ = bil_b.reshape(1, 1)

  p0, r0 = _proj2_aug(user_x, W_l0, W_r0)
  acc0 = _edge_pass_aug(p0, e3)
  h1, dinv = _bn0(acc0, r0, b_l0_, gamma0_, beta0_)
  p1, r1 = _proj2(h1, W_l1, W_r1)
  acc1 = _edge_pass(p1, e3)
  return _final(acc1, dinv, r1, b_l1_, gamma1_, beta1_, ded_table, bil_W,
                bil_b_)
